# baseline (speedup 1.0000x reference)
import jax
import jax.numpy as jnp
from jax.experimental import pallas as pl


_S_BLK = 1024


def _add_kernel(x_ref, e_ref, o_ref):
    s = pl.program_id(0)
    o_ref[...] = x_ref[...] + e_ref[pl.ds(s * _S_BLK, _S_BLK), :]


def kernel(inputs, embeddings):
    batch, seq_len, dim = inputs.shape
    pos = embeddings[:seq_len]
    ns = seq_len // _S_BLK
    return pl.pallas_call(
        _add_kernel,
        grid=(ns, batch),
        in_specs=[
            pl.BlockSpec((1, _S_BLK, dim), lambda s, b: (b, s, 0)),
            pl.BlockSpec((seq_len, dim), lambda s, b: (0, 0)),
        ],
        out_specs=pl.BlockSpec((1, _S_BLK, dim), lambda s, b: (b, s, 0)),
        out_shape=jax.ShapeDtypeStruct(inputs.shape, inputs.dtype),
    )(inputs, pos)
